# Initial kernel scaffold; baseline (speedup 1.0000x reference)
import jax
import jax.numpy as jnp
from jax.experimental import pallas as pl


def kernel(x, edge_attr, W_e1, b_e1, W_e2, b_e2, W_n1, b_n1, W_n2, b_n2, ln_n_scale, ln_n_bias, ln_e_scale, ln_e_bias, edge_index):
    raise NotImplementedError("write your pallas kernel here")


# speedup vs baseline: 2.0190x; 2.0190x over previous
import functools

import jax
import jax.numpy as jnp
from jax import lax
from jax.experimental import pallas as pl
from jax.experimental.pallas import tpu as pltpu
from jax.experimental.pallas import tpu_sc as plsc


def _ln(r, scale, bias, eps=1e-5):
    mu = jnp.mean(r, axis=-1, keepdims=True)
    var = jnp.mean((r - mu) ** 2, axis=-1, keepdims=True)
    return (r - mu) * lax.rsqrt(var + eps) * scale + bias


def _precompute_tables(x, w_sd):
    n, d = x.shape
    bn = 1000
    assert n % bn == 0

    def body(x_ref, w_ref, ps_ref, pd_ref):
        xb = x_ref[...]
        w = w_ref[...]
        ps_ref[...] = jnp.dot(xb, w[:d], preferred_element_type=jnp.float32)
        pd_ref[...] = jnp.dot(xb, w[d:], preferred_element_type=jnp.float32)

    return pl.pallas_call(
        body,
        grid=(n // bn,),
        in_specs=[
            pl.BlockSpec((bn, d), lambda i: (i, 0)),
            pl.BlockSpec((2 * d, d), lambda i: (0, 0)),
        ],
        out_specs=[
            pl.BlockSpec((bn, d), lambda i: (i, 0)),
            pl.BlockSpec((bn, d), lambda i: (i, 0)),
        ],
        out_shape=[
            jax.ShapeDtypeStruct((n, d), jnp.float32),
            jax.ShapeDtypeStruct((n, d), jnp.float32),
        ],
    )(x, w_sd)


def _sc_gather_pair(src, dst, ps, pd):
    e = src.shape[0]
    _, d = ps.shape
    info = plsc.get_sparse_core_info()
    nc, ns = info.num_cores, info.num_subcores
    nw = nc * ns
    ew = e // nw
    ch = 40
    nit = ew // ch
    assert ew % ch == 0 and e % nw == 0

    mesh = plsc.VectorSubcoreMesh(core_axis_name="c", subcore_axis_name="s")

    @functools.partial(
        pl.kernel,
        out_type=(
            jax.ShapeDtypeStruct((e, d), jnp.float32),
            jax.ShapeDtypeStruct((e, d), jnp.float32),
        ),
        mesh=mesh,
        scratch_types=[
            pltpu.VMEM((ch,), jnp.int32),
            pltpu.VMEM((ch,), jnp.int32),
            pltpu.VMEM((ch, d), jnp.float32),
            pltpu.VMEM((ch, d), jnp.float32),
            pltpu.SemaphoreType.DMA,
            pltpu.SemaphoreType.DMA,
        ],
    )
    def k(src_hbm, dst_hbm, ps_hbm, pd_hbm, g1_hbm, g2_hbm,
          idx_a, idx_b, buf_a, buf_b, sem_a, sem_b):
        wid = lax.axis_index("s") * nc + lax.axis_index("c")

        @pl.loop(0, nit)
        def _(i):
            base = wid * ew + i * ch
            pltpu.sync_copy(src_hbm.at[pl.ds(base, ch)], idx_a)
            pltpu.sync_copy(dst_hbm.at[pl.ds(base, ch)], idx_b)
            c1 = pltpu.async_copy(ps_hbm.at[idx_a], buf_a, sem_a)
            c2 = pltpu.async_copy(pd_hbm.at[idx_b], buf_b, sem_b)
            c1.wait()
            c2.wait()
            pltpu.sync_copy(buf_a, g1_hbm.at[pl.ds(base, ch)])
            pltpu.sync_copy(buf_b, g2_hbm.at[pl.ds(base, ch)])

    return k(src, dst, ps, pd)


def _edge_mlp(g1, g2, edge_attr, w_e, b_e1, w_e2, b_e2, ln_s, ln_b):
    e, d = edge_attr.shape
    be = 1000
    assert e % be == 0

    def body(g1_ref, g2_ref, ea_ref, we_ref, b1_ref, w2_ref, b2_ref,
             s_ref, b_ref, u_ref, eo_ref):
        ea = ea_ref[...]
        z = (g1_ref[...] + g2_ref[...] + b1_ref[...]
             + jnp.dot(ea, we_ref[...], preferred_element_type=jnp.float32))
        t = jnp.maximum(z, 0.0)
        u = jnp.dot(t, w2_ref[...], preferred_element_type=jnp.float32) + b2_ref[...]
        u_ref[...] = u
        eo_ref[...] = _ln(u + ea, s_ref[...], b_ref[...])

    vec = lambda: pl.BlockSpec((1, d), lambda i: (0, 0))
    mat = lambda: pl.BlockSpec((d, d), lambda i: (0, 0))
    blk = lambda: pl.BlockSpec((be, d), lambda i: (i, 0))
    return pl.pallas_call(
        body,
        grid=(e // be,),
        in_specs=[blk(), blk(), blk(), mat(), vec(), mat(), vec(), vec(), vec()],
        out_specs=[blk(), blk()],
        out_shape=[
            jax.ShapeDtypeStruct((e, d), jnp.float32),
            jax.ShapeDtypeStruct((e, d), jnp.float32),
        ],
    )(g1, g2, edge_attr, w_e, b_e1.reshape(1, d), w_e2, b_e2.reshape(1, d),
      ln_s.reshape(1, d), ln_b.reshape(1, d))


def _sc_segment_sum(dst, u, zeros_half):
    e, d = u.shape
    n, dh = zeros_half.shape
    assert dh == d // 2
    info = plsc.get_sparse_core_info()
    nc, ns = info.num_cores, info.num_subcores
    et = e // ns
    ch = 40
    nit = et // ch
    rc = 400
    nrc = n // rc
    nround = (nrc + ns - 1) // ns
    assert et % ch == 0 and n % rc == 0

    mesh = plsc.VectorSubcoreMesh(core_axis_name="c", subcore_axis_name="s")

    @functools.partial(
        pl.kernel,
        out_type=jax.ShapeDtypeStruct((n, d), jnp.float32),
        mesh=mesh,
        scratch_types=[
            pltpu.VMEM((ch,), jnp.int32),
            pltpu.VMEM((ch, dh), jnp.float32),
            pltpu.MemorySpace.VMEM_SHARED((n, dh), jnp.float32),
        ],
    )
    def k(dst_hbm, u_hbm, z_hbm, agg_hbm, idxb, rowb, acc):
        cid = lax.axis_index("c")
        sid = lax.axis_index("s")
        col0 = cid * dh

        for r in range(nround):
            j = r * ns + sid

            @pl.when(j < nrc)
            def _():
                pltpu.sync_copy(z_hbm.at[pl.ds(j * rc, rc)],
                                acc.at[pl.ds(j * rc, rc)])

        plsc.subcore_barrier()

        @pl.loop(0, nit)
        def _(i):
            base = sid * et + i * ch
            pltpu.sync_copy(dst_hbm.at[pl.ds(base, ch)], idxb)
            pltpu.sync_copy(u_hbm.at[pl.ds(base, ch), pl.ds(col0, dh)], rowb)
            pltpu.sync_copy(rowb, acc.at[idxb], add=True)

        plsc.subcore_barrier()

        for r in range(nround):
            j = r * ns + sid

            @pl.when(j < nrc)
            def _():
                pltpu.sync_copy(acc.at[pl.ds(j * rc, rc)],
                                agg_hbm.at[pl.ds(j * rc, rc), pl.ds(col0, dh)])

    return k(dst, u, zeros_half)


def _node_mlp(x, agg, w_n1, b_n1, w_n2, b_n2, ln_s, ln_b):
    n, d = x.shape
    bn = 1000
    assert n % bn == 0

    def body(x_ref, a_ref, w1_ref, b1_ref, w2_ref, b2_ref, s_ref, b_ref, o_ref):
        xb = x_ref[...]
        w1 = w1_ref[...]
        z = (jnp.dot(xb, w1[:d], preferred_element_type=jnp.float32)
             + jnp.dot(a_ref[...], w1[d:], preferred_element_type=jnp.float32)
             + b1_ref[...])
        t = jnp.maximum(z, 0.0)
        u = jnp.dot(t, w2_ref[...], preferred_element_type=jnp.float32) + b2_ref[...]
        o_ref[...] = _ln(u + xb, s_ref[...], b_ref[...])

    vec = lambda: pl.BlockSpec((1, d), lambda i: (0, 0))
    blk = lambda: pl.BlockSpec((bn, d), lambda i: (i, 0))
    return pl.pallas_call(
        body,
        grid=(n // bn,),
        in_specs=[blk(), blk(),
                  pl.BlockSpec((2 * d, d), lambda i: (0, 0)), vec(),
                  pl.BlockSpec((d, d), lambda i: (0, 0)), vec(), vec(), vec()],
        out_specs=blk(),
        out_shape=jax.ShapeDtypeStruct((n, d), jnp.float32),
    )(x, agg, w_n1, b_n1.reshape(1, d), w_n2, b_n2.reshape(1, d),
      ln_s.reshape(1, d), ln_b.reshape(1, d))


def kernel(x, edge_attr, W_e1, b_e1, W_e2, b_e2, W_n1, b_n1, W_n2, b_n2,
           ln_n_scale, ln_n_bias, ln_e_scale, ln_e_bias, edge_index):
    n, d = x.shape
    src = edge_index[0]
    dst = edge_index[1]

    ps, pd = _precompute_tables(x, W_e1[: 2 * d])
    g1, g2 = _sc_gather_pair(src, dst, ps, pd)
    u, e_out = _edge_mlp(g1, g2, edge_attr, W_e1[2 * d :], b_e1, W_e2, b_e2,
                         ln_e_scale, ln_e_bias)
    agg = _sc_segment_sum(dst, u, jnp.zeros((n, d // 2), jnp.float32))
    x_out = _node_mlp(x, agg, W_n1, b_n1, W_n2, b_n2, ln_n_scale, ln_n_bias)
    return x_out, e_out


# speedup vs baseline: 2.9395x; 1.4559x over previous
import functools

import jax
import jax.numpy as jnp
from jax import lax
from jax.experimental import pallas as pl
from jax.experimental.pallas import tpu as pltpu
from jax.experimental.pallas import tpu_sc as plsc


def _ln(r, scale, bias, eps=1e-5):
    mu = jnp.mean(r, axis=-1, keepdims=True)
    var = jnp.mean((r - mu) ** 2, axis=-1, keepdims=True)
    return (r - mu) * lax.rsqrt(var + eps) * scale + bias


def _precompute_tables(x, w_sd):
    n, d = x.shape
    bn = 1000
    assert n % bn == 0

    def body(x_ref, w_ref, ps_ref, pd_ref):
        xb = x_ref[...]
        w = w_ref[...]
        ps_ref[...] = jnp.dot(xb, w[:d], preferred_element_type=jnp.float32)
        pd_ref[...] = jnp.dot(xb, w[d:], preferred_element_type=jnp.float32)

    return pl.pallas_call(
        body,
        grid=(n // bn,),
        in_specs=[
            pl.BlockSpec((bn, d), lambda i: (i, 0)),
            pl.BlockSpec((2 * d, d), lambda i: (0, 0)),
        ],
        out_specs=[
            pl.BlockSpec((bn, d), lambda i: (i, 0)),
            pl.BlockSpec((bn, d), lambda i: (i, 0)),
        ],
        out_shape=[
            jax.ShapeDtypeStruct((n, d), jnp.float32),
            jax.ShapeDtypeStruct((n, d), jnp.float32),
        ],
    )(x, w_sd)


def _sc_gather_pair(src, dst, ps, pd):
    e = src.shape[0]
    _, d = ps.shape
    info = plsc.get_sparse_core_info()
    nc, ns = info.num_cores, info.num_subcores
    nw = nc * ns
    ew = e // nw
    ch = 200
    nit = ew // ch
    assert ew % ch == 0 and e % nw == 0

    mesh = plsc.VectorSubcoreMesh(core_axis_name="c", subcore_axis_name="s")

    @functools.partial(
        pl.kernel,
        out_type=(
            jax.ShapeDtypeStruct((e, d), jnp.float32),
            jax.ShapeDtypeStruct((e, d), jnp.float32),
        ),
        mesh=mesh,
        scratch_types=[
            pltpu.VMEM((ch,), jnp.int32),
            pltpu.VMEM((ch,), jnp.int32),
            pltpu.VMEM((ch, d), jnp.float32),
            pltpu.VMEM((ch, d), jnp.float32),
            pltpu.SemaphoreType.DMA,
            pltpu.SemaphoreType.DMA,
        ],
    )
    def k(src_hbm, dst_hbm, ps_hbm, pd_hbm, g1_hbm, g2_hbm,
          idx_a, idx_b, buf_a, buf_b, sem_a, sem_b):
        wid = lax.axis_index("s") * nc + lax.axis_index("c")

        @pl.loop(0, nit)
        def _(i):
            base = wid * ew + i * ch
            pltpu.sync_copy(src_hbm.at[pl.ds(base, ch)], idx_a)
            pltpu.sync_copy(dst_hbm.at[pl.ds(base, ch)], idx_b)
            c1 = pltpu.async_copy(ps_hbm.at[idx_a], buf_a, sem_a)
            c2 = pltpu.async_copy(pd_hbm.at[idx_b], buf_b, sem_b)
            c1.wait()
            c2.wait()
            pltpu.sync_copy(buf_a, g1_hbm.at[pl.ds(base, ch)])
            pltpu.sync_copy(buf_b, g2_hbm.at[pl.ds(base, ch)])

    return k(src, dst, ps, pd)


def _edge_mlp(g1, g2, edge_attr, w_e, b_e1, w_e2, b_e2, ln_s, ln_b):
    e, d = edge_attr.shape
    be = 1000
    assert e % be == 0

    def body(g1_ref, g2_ref, ea_ref, we_ref, b1_ref, w2_ref, b2_ref,
             s_ref, b_ref, u_ref, eo_ref):
        ea = ea_ref[...]
        z = (g1_ref[...] + g2_ref[...] + b1_ref[...]
             + jnp.dot(ea, we_ref[...], preferred_element_type=jnp.float32))
        t = jnp.maximum(z, 0.0)
        u = jnp.dot(t, w2_ref[...], preferred_element_type=jnp.float32) + b2_ref[...]
        u_ref[...] = u
        eo_ref[...] = _ln(u + ea, s_ref[...], b_ref[...])

    vec = lambda: pl.BlockSpec((1, d), lambda i: (0, 0))
    mat = lambda: pl.BlockSpec((d, d), lambda i: (0, 0))
    blk = lambda: pl.BlockSpec((be, d), lambda i: (i, 0))
    return pl.pallas_call(
        body,
        grid=(e // be,),
        in_specs=[blk(), blk(), blk(), mat(), vec(), mat(), vec(), vec(), vec()],
        out_specs=[blk(), blk()],
        out_shape=[
            jax.ShapeDtypeStruct((e, d), jnp.float32),
            jax.ShapeDtypeStruct((e, d), jnp.float32),
        ],
    )(g1, g2, edge_attr, w_e, b_e1.reshape(1, d), w_e2, b_e2.reshape(1, d),
      ln_s.reshape(1, d), ln_b.reshape(1, d))


def _sc_segment_sum(dst, u, zeros_half):
    e, d = u.shape
    n, dh = zeros_half.shape
    assert dh == d // 2
    info = plsc.get_sparse_core_info()
    nc, ns = info.num_cores, info.num_subcores
    et = e // ns
    ch = 200
    nit = et // ch
    rc = 400
    nrc = n // rc
    nround = (nrc + ns - 1) // ns
    assert et % ch == 0 and n % rc == 0

    mesh = plsc.VectorSubcoreMesh(core_axis_name="c", subcore_axis_name="s")

    @functools.partial(
        pl.kernel,
        out_type=jax.ShapeDtypeStruct((n, d), jnp.float32),
        mesh=mesh,
        scratch_types=[
            pltpu.VMEM((ch,), jnp.int32),
            pltpu.VMEM((ch, dh), jnp.float32),
            pltpu.MemorySpace.VMEM_SHARED((n, dh), jnp.float32),
        ],
    )
    def k(dst_hbm, u_hbm, z_hbm, agg_hbm, idxb, rowb, acc):
        cid = lax.axis_index("c")
        sid = lax.axis_index("s")
        col0 = cid * dh

        for r in range(nround):
            j = r * ns + sid

            @pl.when(j < nrc)
            def _():
                pltpu.sync_copy(z_hbm.at[pl.ds(j * rc, rc)],
                                acc.at[pl.ds(j * rc, rc)])

        plsc.subcore_barrier()

        @pl.loop(0, nit)
        def _(i):
            base = sid * et + i * ch
            pltpu.sync_copy(dst_hbm.at[pl.ds(base, ch)], idxb)
            pltpu.sync_copy(u_hbm.at[pl.ds(base, ch), pl.ds(col0, dh)], rowb)
            pltpu.sync_copy(rowb, acc.at[idxb], add=True)

        plsc.subcore_barrier()

        for r in range(nround):
            j = r * ns + sid

            @pl.when(j < nrc)
            def _():
                pltpu.sync_copy(acc.at[pl.ds(j * rc, rc)],
                                agg_hbm.at[pl.ds(j * rc, rc), pl.ds(col0, dh)])

    return k(dst, u, zeros_half)


def _node_mlp(x, agg, w_n1, b_n1, w_n2, b_n2, ln_s, ln_b):
    n, d = x.shape
    bn = 1000
    assert n % bn == 0

    def body(x_ref, a_ref, w1_ref, b1_ref, w2_ref, b2_ref, s_ref, b_ref, o_ref):
        xb = x_ref[...]
        w1 = w1_ref[...]
        z = (jnp.dot(xb, w1[:d], preferred_element_type=jnp.float32)
             + jnp.dot(a_ref[...], w1[d:], preferred_element_type=jnp.float32)
             + b1_ref[...])
        t = jnp.maximum(z, 0.0)
        u = jnp.dot(t, w2_ref[...], preferred_element_type=jnp.float32) + b2_ref[...]
        o_ref[...] = _ln(u + xb, s_ref[...], b_ref[...])

    vec = lambda: pl.BlockSpec((1, d), lambda i: (0, 0))
    blk = lambda: pl.BlockSpec((bn, d), lambda i: (i, 0))
    return pl.pallas_call(
        body,
        grid=(n // bn,),
        in_specs=[blk(), blk(),
                  pl.BlockSpec((2 * d, d), lambda i: (0, 0)), vec(),
                  pl.BlockSpec((d, d), lambda i: (0, 0)), vec(), vec(), vec()],
        out_specs=blk(),
        out_shape=jax.ShapeDtypeStruct((n, d), jnp.float32),
    )(x, agg, w_n1, b_n1.reshape(1, d), w_n2, b_n2.reshape(1, d),
      ln_s.reshape(1, d), ln_b.reshape(1, d))


def kernel(x, edge_attr, W_e1, b_e1, W_e2, b_e2, W_n1, b_n1, W_n2, b_n2,
           ln_n_scale, ln_n_bias, ln_e_scale, ln_e_bias, edge_index):
    n, d = x.shape
    src = edge_index[0]
    dst = edge_index[1]

    ps, pd = _precompute_tables(x, W_e1[: 2 * d])
    g1, g2 = _sc_gather_pair(src, dst, ps, pd)
    u, e_out = _edge_mlp(g1, g2, edge_attr, W_e1[2 * d :], b_e1, W_e2, b_e2,
                         ln_e_scale, ln_e_bias)
    agg = _sc_segment_sum(dst, u, jnp.zeros((n, d // 2), jnp.float32))
    x_out = _node_mlp(x, agg, W_n1, b_n1, W_n2, b_n2, ln_n_scale, ln_n_bias)
    return x_out, e_out


# speedup vs baseline: 3.3089x; 1.1257x over previous
import functools

import jax
import jax.numpy as jnp
from jax import lax
from jax.experimental import pallas as pl
from jax.experimental.pallas import tpu as pltpu
from jax.experimental.pallas import tpu_sc as plsc


def _ln(r, scale, bias, eps=1e-5):
    mu = jnp.mean(r, axis=-1, keepdims=True)
    var = jnp.mean((r - mu) ** 2, axis=-1, keepdims=True)
    return (r - mu) * lax.rsqrt(var + eps) * scale + bias


def _precompute_tables(x, w_sd):
    n, d = x.shape
    bn = 1000
    assert n % bn == 0

    def body(x_ref, w_ref, ps_ref, pd_ref):
        xb = x_ref[...]
        w = w_ref[...]
        ps_ref[...] = jnp.dot(xb, w[:d], preferred_element_type=jnp.float32)
        pd_ref[...] = jnp.dot(xb, w[d:], preferred_element_type=jnp.float32)

    return pl.pallas_call(
        body,
        grid=(n // bn,),
        in_specs=[
            pl.BlockSpec((bn, d), lambda i: (i, 0)),
            pl.BlockSpec((2 * d, d), lambda i: (0, 0)),
        ],
        out_specs=[
            pl.BlockSpec((bn, d), lambda i: (i, 0)),
            pl.BlockSpec((bn, d), lambda i: (i, 0)),
        ],
        out_shape=[
            jax.ShapeDtypeStruct((n, d), jnp.float32),
            jax.ShapeDtypeStruct((n, d), jnp.float32),
        ],
    )(x, w_sd)


def _sc_gather_pair(src, dst, ps, pd):
    e = src.shape[0]
    _, d = ps.shape
    info = plsc.get_sparse_core_info()
    nc, ns = info.num_cores, info.num_subcores
    nw = nc * ns
    ew = e // nw
    ch = 112
    nfull = (ew // ch) & ~1
    tail = ew - nfull * ch
    assert e % nw == 0 and tail % 8 == 0

    mesh = plsc.VectorSubcoreMesh(core_axis_name="c", subcore_axis_name="s")

    @functools.partial(
        pl.kernel,
        out_type=(
            jax.ShapeDtypeStruct((e, d), jnp.float32),
            jax.ShapeDtypeStruct((e, d), jnp.float32),
        ),
        mesh=mesh,
        scratch_types=[
            pltpu.VMEM((ew,), jnp.int32),
            pltpu.VMEM((ew,), jnp.int32),
            [pltpu.VMEM((ch, d), jnp.float32) for _ in range(2)],
            [pltpu.VMEM((ch, d), jnp.float32) for _ in range(2)],
            [pltpu.SemaphoreType.DMA for _ in range(2)],
            [pltpu.SemaphoreType.DMA for _ in range(2)],
        ],
    )
    def k(src_hbm, dst_hbm, ps_hbm, pd_hbm, g1_hbm, g2_hbm,
          idx_s, idx_d, bufs_a, bufs_b, sems_g, sems_w):
        wid = lax.axis_index("s") * nc + lax.axis_index("c")
        base = wid * ew
        pltpu.sync_copy(src_hbm.at[pl.ds(base, ew)], idx_s)
        pltpu.sync_copy(dst_hbm.at[pl.ds(base, ew)], idx_d)

        def fire_gather(i, b, n):
            pltpu.async_copy(ps_hbm.at[idx_s.at[pl.ds(i * ch, n)]],
                             bufs_a[b].at[pl.ds(0, n)], sems_g[b])
            pltpu.async_copy(pd_hbm.at[idx_d.at[pl.ds(i * ch, n)]],
                             bufs_b[b].at[pl.ds(0, n)], sems_g[b])

        def wait_gather(i, b, n):
            pltpu.make_async_copy(ps_hbm.at[idx_s.at[pl.ds(i * ch, n)]],
                                  bufs_a[b].at[pl.ds(0, n)], sems_g[b]).wait()
            pltpu.make_async_copy(pd_hbm.at[idx_d.at[pl.ds(i * ch, n)]],
                                  bufs_b[b].at[pl.ds(0, n)], sems_g[b]).wait()

        def fire_wb(i, b, n):
            row0 = base + i * ch
            pltpu.async_copy(bufs_a[b].at[pl.ds(0, n)],
                             g1_hbm.at[pl.ds(row0, n)], sems_w[b])
            pltpu.async_copy(bufs_b[b].at[pl.ds(0, n)],
                             g2_hbm.at[pl.ds(row0, n)], sems_w[b])

        def wait_wb(i, b, n):
            row0 = base + i * ch
            pltpu.make_async_copy(bufs_a[b].at[pl.ds(0, n)],
                                  g1_hbm.at[pl.ds(row0, n)], sems_w[b]).wait()
            pltpu.make_async_copy(bufs_b[b].at[pl.ds(0, n)],
                                  g2_hbm.at[pl.ds(row0, n)], sems_w[b]).wait()

        @pl.loop(0, nfull // 2)
        def _(g):
            for b in (0, 1):
                i = g * 2 + b

                @pl.when(i >= 2)
                def _():
                    wait_wb(i - 2, b, ch)

                fire_gather(i, b, ch)
            for b in (0, 1):
                i = g * 2 + b
                wait_gather(i, b, ch)
                fire_wb(i, b, ch)

        if nfull >= 2:
            wait_wb(nfull - 2, 0, ch)
            wait_wb(nfull - 1, 1, ch)
        if tail:
            fire_gather(nfull, 0, tail)
            wait_gather(nfull, 0, tail)
            fire_wb(nfull, 0, tail)
            wait_wb(nfull, 0, tail)

    return k(src, dst, ps, pd)


def _edge_mlp(g1, g2, edge_attr, w_e, b_e1, w_e2, b_e2, ln_s, ln_b):
    e, d = edge_attr.shape
    be = 1000
    assert e % be == 0

    def body(g1_ref, g2_ref, ea_ref, we_ref, b1_ref, w2_ref, b2_ref,
             s_ref, b_ref, u_ref, eo_ref):
        ea = ea_ref[...]
        z = (g1_ref[...] + g2_ref[...] + b1_ref[...]
             + jnp.dot(ea, we_ref[...], preferred_element_type=jnp.float32))
        t = jnp.maximum(z, 0.0)
        u = jnp.dot(t, w2_ref[...], preferred_element_type=jnp.float32) + b2_ref[...]
        u_ref[...] = u
        eo_ref[...] = _ln(u + ea, s_ref[...], b_ref[...])

    vec = lambda: pl.BlockSpec((1, d), lambda i: (0, 0))
    mat = lambda: pl.BlockSpec((d, d), lambda i: (0, 0))
    blk = lambda: pl.BlockSpec((be, d), lambda i: (i, 0))
    return pl.pallas_call(
        body,
        grid=(e // be,),
        in_specs=[blk(), blk(), blk(), mat(), vec(), mat(), vec(), vec(), vec()],
        out_specs=[blk(), blk()],
        out_shape=[
            jax.ShapeDtypeStruct((e, d), jnp.float32),
            jax.ShapeDtypeStruct((e, d), jnp.float32),
        ],
    )(g1, g2, edge_attr, w_e, b_e1.reshape(1, d), w_e2, b_e2.reshape(1, d),
      ln_s.reshape(1, d), ln_b.reshape(1, d))


def _sc_segment_sum(dst, u, zeros_half):
    e, d = u.shape
    n, dh = zeros_half.shape
    assert dh == d // 2
    info = plsc.get_sparse_core_info()
    nc, ns = info.num_cores, info.num_subcores
    et = e // ns
    ch = 80
    nit = et // ch
    rc = 400
    nrc = n // rc
    nround = (nrc + ns - 1) // ns
    assert et % ch == 0 and n % rc == 0

    mesh = plsc.VectorSubcoreMesh(core_axis_name="c", subcore_axis_name="s")

    @functools.partial(
        pl.kernel,
        out_type=jax.ShapeDtypeStruct((n, d), jnp.float32),
        mesh=mesh,
        scratch_types=[
            pltpu.VMEM((nit, ch), jnp.int32),
            [pltpu.VMEM((ch, dh), jnp.float32) for _ in range(2)],
            [pltpu.SemaphoreType.DMA for _ in range(2)],
            pltpu.MemorySpace.VMEM_SHARED((n, dh), jnp.float32),
        ],
    )
    def k(dst2_hbm, u_hbm, z_hbm, agg_hbm, idx2, rows, sems, acc):
        cid = lax.axis_index("c")
        sid = lax.axis_index("s")
        col0 = cid * dh

        def fire_rows(i, b):
            pltpu.async_copy(
                u_hbm.at[pl.ds(sid * et + i * ch, ch), pl.ds(col0, dh)],
                rows[b], sems[b])

        def wait_rows(i, b):
            pltpu.make_async_copy(
                u_hbm.at[pl.ds(sid * et + i * ch, ch), pl.ds(col0, dh)],
                rows[b], sems[b]).wait()

        pltpu.sync_copy(dst2_hbm.at[sid], idx2)

        for r in range(nround):
            j = r * ns + sid

            @pl.when(j < nrc)
            def _():
                pltpu.sync_copy(z_hbm.at[pl.ds(j * rc, rc)],
                                acc.at[pl.ds(j * rc, rc)])

        fire_rows(0, 0)
        plsc.subcore_barrier()

        @pl.loop(0, nit // 2)
        def _(g):
            for b in (0, 1):
                i = g * 2 + b

                @pl.when(i + 1 < nit)
                def _():
                    fire_rows(i + 1, 1 - b)

                wait_rows(i, b)
                pltpu.sync_copy(rows[b], acc.at[idx2.at[i]], add=True)

        if nit % 2:
            i = nit - 1
            wait_rows(i, i % 2)
            pltpu.sync_copy(rows[i % 2], acc.at[idx2.at[i]], add=True)

        plsc.subcore_barrier()

        for r in range(nround):
            j = r * ns + sid

            @pl.when(j < nrc)
            def _():
                pltpu.sync_copy(acc.at[pl.ds(j * rc, rc)],
                                agg_hbm.at[pl.ds(j * rc, rc), pl.ds(col0, dh)])

    return k(dst.reshape(ns, nit, ch), u, zeros_half)


def _node_mlp(x, agg, w_n1, b_n1, w_n2, b_n2, ln_s, ln_b):
    n, d = x.shape
    bn = 1000
    assert n % bn == 0

    def body(x_ref, a_ref, w1_ref, b1_ref, w2_ref, b2_ref, s_ref, b_ref, o_ref):
        xb = x_ref[...]
        w1 = w1_ref[...]
        z = (jnp.dot(xb, w1[:d], preferred_element_type=jnp.float32)
             + jnp.dot(a_ref[...], w1[d:], preferred_element_type=jnp.float32)
             + b1_ref[...])
        t = jnp.maximum(z, 0.0)
        u = jnp.dot(t, w2_ref[...], preferred_element_type=jnp.float32) + b2_ref[...]
        o_ref[...] = _ln(u + xb, s_ref[...], b_ref[...])

    vec = lambda: pl.BlockSpec((1, d), lambda i: (0, 0))
    blk = lambda: pl.BlockSpec((bn, d), lambda i: (i, 0))
    return pl.pallas_call(
        body,
        grid=(n // bn,),
        in_specs=[blk(), blk(),
                  pl.BlockSpec((2 * d, d), lambda i: (0, 0)), vec(),
                  pl.BlockSpec((d, d), lambda i: (0, 0)), vec(), vec(), vec()],
        out_specs=blk(),
        out_shape=jax.ShapeDtypeStruct((n, d), jnp.float32),
    )(x, agg, w_n1, b_n1.reshape(1, d), w_n2, b_n2.reshape(1, d),
      ln_s.reshape(1, d), ln_b.reshape(1, d))


def kernel(x, edge_attr, W_e1, b_e1, W_e2, b_e2, W_n1, b_n1, W_n2, b_n2,
           ln_n_scale, ln_n_bias, ln_e_scale, ln_e_bias, edge_index):
    n, d = x.shape
    src = edge_index[0]
    dst = edge_index[1]

    ps, pd = _precompute_tables(x, W_e1[: 2 * d])
    g1, g2 = _sc_gather_pair(src, dst, ps, pd)
    u, e_out = _edge_mlp(g1, g2, edge_attr, W_e1[2 * d :], b_e1, W_e2, b_e2,
                         ln_e_scale, ln_e_bias)
    agg = _sc_segment_sum(dst, u, jnp.zeros((n, d // 2), jnp.float32))
    x_out = _node_mlp(x, agg, W_n1, b_n1, W_n2, b_n2, ln_n_scale, ln_n_bias)
    return x_out, e_out
